# Initial kernel scaffold; baseline (speedup 1.0000x reference)
import functools

import jax
import jax.numpy as jnp
from jax import lax
from jax.experimental import pallas as pl
from jax.experimental.pallas import tpu as pltpu
from jax.experimental.pallas import tpu_sc as plsc

NUM_EMBED = 8
NUM_EMBEDDING = 256
EMBED_DIM = 64

NC = 2
NS = 16
NW = NC * NS

LANES = 16
CHUNK = 128


def _build(batch):
    total_rows = batch * NUM_EMBED
    rows_per_w = total_rows // NW
    n_per_w = batch // NW
    n_chunks = rows_per_w // CHUNK

    mesh = plsc.VectorSubcoreMesh(
        core_axis_name="c", subcore_axis_name="s", num_cores=NC,
        num_subcores=NS)

    @functools.partial(
        pl.kernel,
        out_type=jax.ShapeDtypeStruct((total_rows, EMBED_DIM), jnp.float32),
        mesh=mesh,
        scratch_types=[
            pltpu.VMEM((n_per_w,), jnp.int32),
            pltpu.VMEM((n_chunks, CHUNK), jnp.int32),
            pltpu.VMEM((CHUNK, EMBED_DIM), jnp.float32),
            pltpu.SemaphoreType.DMA,
        ],
    )
    def k(x_hbm, tab_hbm, out_hbm, x_v, idx_v, rows_v, sem):
        wid = lax.axis_index("s") * NC + lax.axis_index("c")
        nbase = wid * n_per_w
        rbase = wid * rows_per_w

        pltpu.sync_copy(x_hbm.at[pl.ds(nbase, n_per_w)], x_v)

        lane = lax.iota(jnp.int32, 16)
        nsel = lax.shift_right_logical(lane, 3)
        shiftv = lax.shift_left(lane & 3, 3)
        basev = lax.shift_left(lane & 7, 8)

        def compute(j, _):
            for c in range(8):
                i = j * 8 + c
                x = plsc.load_gather(x_v, [nsel + 2 * i])
                byte = lax.shift_right_logical(jnp.abs(x), shiftv) & 255
                idx_v[j, pl.ds(c * LANES, LANES)] = basev + byte
            return 0

        lax.fori_loop(0, n_chunks, compute, 0)

        def gather(j, _):
            pltpu.async_copy(tab_hbm.at[idx_v.at[j]], rows_v, sem).wait()
            pltpu.sync_copy(
                rows_v, out_hbm.at[pl.ds(rbase + j * CHUNK, CHUNK)])
            return 0

        lax.fori_loop(0, n_chunks, gather, 0)

    return k


@jax.jit
def kernel(X, tables):
    batch = X.shape[0]
    tab2d = tables.reshape(NUM_EMBED * NUM_EMBEDDING, EMBED_DIM)
    out = _build(batch)(X, tab2d)
    return out.reshape(batch, NUM_EMBED * EMBED_DIM)


# speedup vs baseline: 4.5431x; 4.5431x over previous
import functools

import jax
import jax.numpy as jnp
from jax import lax
from jax.experimental import pallas as pl
from jax.experimental.pallas import tpu as pltpu
from jax.experimental.pallas import tpu_sc as plsc

NUM_EMBED = 8
NUM_EMBEDDING = 256
EMBED_DIM = 64

NC = 2
NS = 16
NW = NC * NS

LANES = 16
CHUNK = 128


def _build(batch):
    total_rows = batch * NUM_EMBED
    rows_per_w = total_rows // NW
    n_per_w = batch // NW
    n_chunks = rows_per_w // CHUNK

    mesh = plsc.VectorSubcoreMesh(
        core_axis_name="c", subcore_axis_name="s", num_cores=NC,
        num_subcores=NS)

    @functools.partial(
        pl.kernel,
        out_type=jax.ShapeDtypeStruct((total_rows, EMBED_DIM), jnp.float32),
        mesh=mesh,
        compiler_params=pltpu.CompilerParams(
            needs_layout_passes=False, use_tc_tiling_on_sc=False),
        scratch_types=[
            pltpu.VMEM((n_per_w,), jnp.int32),
            pltpu.VMEM((n_chunks, CHUNK), jnp.int32),
            pltpu.VMEM((CHUNK, EMBED_DIM), jnp.float32),
            pltpu.SemaphoreType.DMA,
        ],
    )
    def k(x_hbm, tab_hbm, out_hbm, x_v, idx_v, rows_v, sem):
        wid = lax.axis_index("s") * NC + lax.axis_index("c")
        nbase = wid * n_per_w
        rbase = wid * rows_per_w

        pltpu.sync_copy(x_hbm.at[pl.ds(nbase, n_per_w)], x_v)

        lane = lax.iota(jnp.int32, 16)
        nsel = lax.shift_right_logical(lane, 3)
        shiftv = lax.shift_left(lane & 3, 3)
        basev = lax.shift_left(lane & 7, 8)

        def compute(j, _):
            for c in range(8):
                i = j * 8 + c
                x = plsc.load_gather(x_v, [nsel + 2 * i])
                byte = lax.shift_right_logical(jnp.abs(x), shiftv) & 255
                idx_v[j, pl.ds(c * LANES, LANES)] = basev + byte
            return 0

        lax.fori_loop(0, n_chunks, compute, 0)

        def gather(j, _):
            pltpu.async_copy(tab_hbm.at[idx_v.at[j]], rows_v, sem).wait()
            pltpu.sync_copy(
                rows_v, out_hbm.at[pl.ds(rbase + j * CHUNK, CHUNK)])
            return 0

        lax.fori_loop(0, n_chunks, gather, 0)

    return k


@jax.jit
def kernel(X, tables):
    batch = X.shape[0]
    tab2d = tables.reshape(NUM_EMBED * NUM_EMBEDDING, EMBED_DIM)
    out = _build(batch)(X, tab2d)
    return out.reshape(batch, NUM_EMBED * EMBED_DIM)
